# baseline (speedup 1.0000x reference)
import functools

import jax
import jax.numpy as jnp
from jax import lax
from jax.experimental import pallas as pl
from jax.experimental.pallas import tpu as pltpu
from jax.experimental.pallas import tpu_sc as plsc

_N = 4194304
_G = 16
_NC = 2
_NS = 16
_NW = _NC * _NS
_PER_TILE = _N // _NW
_L = 16
_CHUNK = 16384
_NCHUNKS = _PER_TILE // _CHUNK
_NPAIR = _NCHUNKS // 2

_mesh = plsc.VectorSubcoreMesh(core_axis_name="c", subcore_axis_name="s")

_scratch = (
    [pltpu.VMEM((_G,), jnp.float32)] * 2
    + [pltpu.VMEM((8 + _G,), jnp.int32)]
    + [pltpu.VMEM((_CHUNK,), jnp.float32)] * 2
    + [pltpu.VMEM((_CHUNK,), jnp.int32)] * 2
    + [pltpu.VMEM((_CHUNK,), jnp.float32)] * 2
    + [pltpu.SemaphoreType.DMA] * 4
)


def _round_bf16_bits(bits):
    lsb = lax.shift_right_logical(bits, 16) & 1
    return (bits + 0x7FFF + lsb) & jnp.int32(-65536)


@functools.partial(
    pl.kernel,
    out_type=jax.ShapeDtypeStruct((_N,), jnp.float32),
    mesh=_mesh,
    scratch_types=_scratch,
    compiler_params=pltpu.CompilerParams(needs_layout_passes=False),
)
def _standardize_sc(x_hbm, g_hbm, c_hbm, s_hbm, out_hbm,
                    c_tab, s_tab, p_tab,
                    x0, x1, g0, g1, o0, o1, si0, si1, so0, so1):
    xb, gb, ob = [x0, x1], [g0, g1], [o0, o1]
    si, so = [si0, si1], [so0, so1]

    wid = lax.axis_index("s") * _NC + lax.axis_index("c")
    base = wid * _PER_TILE

    def start_in(off, b, sz=_CHUNK):
        pltpu.async_copy(x_hbm.at[pl.ds(off, sz)], xb[b].at[pl.ds(0, sz)], si[b])
        pltpu.async_copy(g_hbm.at[pl.ds(off, sz)], gb[b].at[pl.ds(0, sz)], si[b])

    def wait_in(off, b, sz=_CHUNK):
        pltpu.make_async_copy(
            x_hbm.at[pl.ds(off, sz)], xb[b].at[pl.ds(0, sz)], si[b]).wait()
        pltpu.make_async_copy(
            g_hbm.at[pl.ds(off, sz)], gb[b].at[pl.ds(0, sz)], si[b]).wait()

    def start_out(off, b, sz=_CHUNK):
        pltpu.async_copy(ob[b].at[pl.ds(0, sz)], out_hbm.at[pl.ds(off, sz)], so[b])

    def wait_out(off, b, sz=_CHUNK):
        pltpu.make_async_copy(
            ob[b].at[pl.ds(0, sz)], out_hbm.at[pl.ds(off, sz)], so[b]).wait()

    _H = _CHUNK // 2
    start_in(base, 0, _H)
    start_in(base + _H, 1, _H)

    pltpu.sync_copy(c_hbm, c_tab)
    pltpu.sync_copy(s_hbm, s_tab)
    a = 1.0 / s_tab[...]
    b = -(c_tab[...] * a)
    a_hi = _round_bf16_bits(lax.bitcast_convert_type(a, jnp.int32))
    b_hi = _round_bf16_bits(lax.bitcast_convert_type(b, jnp.int32))
    packed = a_hi | lax.shift_right_logical(b_hi, 16)
    plsc.store_scatter(p_tab, [lax.iota(jnp.int32, _L) + 1], packed)

    def compute(b, sz=_CHUNK):
        @plsc.parallel_loop(0, sz, _L, unroll=8)
        def _vec(i, _xv=xb[b], _gv=gb[b], _ov=ob[b]):
            p = plsc.load_gather(p_tab, [_gv[pl.ds(i, _L)]])
            av = lax.bitcast_convert_type(p & jnp.int32(-65536), jnp.float32)
            bv = lax.bitcast_convert_type(lax.shift_left(p, 16), jnp.float32)
            _ov[pl.ds(i, _L)] = _xv[pl.ds(i, _L)] * av + bv

    _F0 = base + _CHUNK

    wait_in(base, 0, _H)
    compute(0, _H)
    start_out(base, 0, _H)
    start_in(_F0, 0)

    wait_in(base + _H, 1, _H)
    compute(1, _H)
    start_out(base + _H, 1, _H)
    start_in(_F0 + _CHUNK, 1)

    def pair_body(j, carry):
        off0 = _F0 + j * (2 * _CHUNK)
        for b in (0, 1):
            off = off0 + b * _CHUNK
            wait_in(off, b)

            @pl.when(j > 0)
            def _():
                wait_out(off - 2 * _CHUNK, b)

            @pl.when(j == 0)
            def _():
                wait_out(base + b * _H, b, _H)

            compute(b)
            start_out(off, b)
            if b == 0:
                start_in(off + 2 * _CHUNK, b)
            else:

                @pl.when(j < 2)
                def _():
                    start_in(off + 2 * _CHUNK, b)

        return carry

    lax.fori_loop(0, 3, pair_body, 0, unroll=False)

    f6 = _F0 + 6 * _CHUNK
    wait_in(f6, 0)
    wait_out(f6 - 2 * _CHUNK, 0)
    compute(0)
    start_out(f6, 0)

    wait_out(f6 - _CHUNK, 1)
    wait_out(f6, 0)


def kernel(x, group, centers, scales):
    return _standardize_sc(x, group, centers, scales)


# speedup vs baseline: 1.0312x; 1.0312x over previous
import functools

import jax
import jax.numpy as jnp
from jax import lax
from jax.experimental import pallas as pl
from jax.experimental.pallas import tpu as pltpu
from jax.experimental.pallas import tpu_sc as plsc

_N = 4194304
_G = 16
_NC = 2
_NS = 16
_NW = _NC * _NS
_PER_TILE = _N // _NW
_L = 16
_CHUNK = 16384
_NCHUNKS = _PER_TILE // _CHUNK
_NPAIR = _NCHUNKS // 2

_mesh = plsc.VectorSubcoreMesh(core_axis_name="c", subcore_axis_name="s")

_scratch = (
    [pltpu.VMEM((_G,), jnp.float32)] * 2
    + [pltpu.VMEM((8 + _G,), jnp.int32)]
    + [pltpu.VMEM((_CHUNK,), jnp.float32)] * 2
    + [pltpu.VMEM((_CHUNK,), jnp.int32)] * 2
    + [pltpu.VMEM((_CHUNK,), jnp.float32)] * 2
    + [pltpu.SemaphoreType.DMA] * 4
)


def _round_bf16_bits(bits):
    lsb = lax.shift_right_logical(bits, 16) & 1
    return (bits + 0x7FFF + lsb) & jnp.int32(-65536)


@functools.partial(
    pl.kernel,
    out_type=jax.ShapeDtypeStruct((_N,), jnp.float32),
    mesh=_mesh,
    scratch_types=_scratch,
    compiler_params=pltpu.CompilerParams(needs_layout_passes=False),
)
def _standardize_sc(x_hbm, g_hbm, c_hbm, s_hbm, out_hbm,
                    c_tab, s_tab, p_tab,
                    x0, x1, g0, g1, o0, o1, si0, si1, so0, so1):
    xb, gb, ob = [x0, x1], [g0, g1], [o0, o1]
    si, so = [si0, si1], [so0, so1]

    wid = lax.axis_index("s") * _NC + lax.axis_index("c")
    base = wid * _PER_TILE

    def start_in(off, b):
        pltpu.async_copy(x_hbm.at[pl.ds(off, _CHUNK)], xb[b], si[b])
        pltpu.async_copy(g_hbm.at[pl.ds(off, _CHUNK)], gb[b], si[b])

    def wait_in(off, b):
        pltpu.make_async_copy(x_hbm.at[pl.ds(off, _CHUNK)], xb[b], si[b]).wait()
        pltpu.make_async_copy(g_hbm.at[pl.ds(off, _CHUNK)], gb[b], si[b]).wait()

    def start_out(off, b):
        pltpu.async_copy(ob[b], out_hbm.at[pl.ds(off, _CHUNK)], so[b])

    def wait_out(off, b):
        pltpu.make_async_copy(ob[b], out_hbm.at[pl.ds(off, _CHUNK)], so[b]).wait()

    start_in(base, 0)
    start_in(base + _CHUNK, 1)

    pltpu.sync_copy(c_hbm, c_tab)
    pltpu.sync_copy(s_hbm, s_tab)
    a = 1.0 / s_tab[...]
    b = -(c_tab[...] * a)
    a_hi = _round_bf16_bits(lax.bitcast_convert_type(a, jnp.int32))
    b_hi = _round_bf16_bits(lax.bitcast_convert_type(b, jnp.int32))
    packed = a_hi | lax.shift_right_logical(b_hi, 16)
    plsc.store_scatter(p_tab, [lax.iota(jnp.int32, _L) + 1], packed)

    def compute(b):
        @plsc.parallel_loop(0, _CHUNK, _L, unroll=8)
        def _vec(i, _xv=xb[b], _gv=gb[b], _ov=ob[b]):
            p = plsc.load_gather(p_tab, [_gv[pl.ds(i, _L)]])
            av = lax.bitcast_convert_type(p & jnp.int32(-65536), jnp.float32)
            bv = lax.bitcast_convert_type(lax.shift_left(p, 16), jnp.float32)
            _ov[pl.ds(i, _L)] = _xv[pl.ds(i, _L)] * av + bv

    def pair_body(j, carry):
        off0 = base + j * (2 * _CHUNK)
        for b in (0, 1):
            off = off0 + b * _CHUNK
            wait_in(off, b)

            @pl.when(j > 0)
            def _():
                wait_out(off - 2 * _CHUNK, b)

            compute(b)
            start_out(off, b)

            @pl.when(j < _NPAIR - 1)
            def _():
                start_in(off + 2 * _CHUNK, b)

        return carry

    lax.fori_loop(0, _NPAIR, pair_body, 0, unroll=False)

    last = base + (_NCHUNKS - 2) * _CHUNK
    wait_out(last, 0)
    wait_out(last + _CHUNK, 1)


def kernel(x, group, centers, scales):
    return _standardize_sc(x, group, centers, scales)
